# initial kernel scaffold (unmeasured)
import jax
import jax.numpy as jnp
from jax import lax
from jax.experimental import pallas as pl
from jax.experimental.pallas import tpu as pltpu

N_DEV = 4
B, SQ, D = 4, 256, 1024
HQ_LOCAL, DH, GROUP = 8, 128, 4
KV_COLS = (HQ_LOCAL // GROUP) * DH
SCALE = 0.08838834764831843


def kernel(x, Wq, Wo, Wk, Wv):
    my_pos = lax.axis_index("i")
    wk_loc = lax.dynamic_slice(Wk, (0, my_pos * KV_COLS), (D, KV_COLS))
    wv_loc = lax.dynamic_slice(Wv, (0, my_pos * KV_COLS), (D, KV_COLS))

    def body(x_ref, wq_ref, wo_ref, wk_ref, wv_ref, out_ref,
             attn_ref, comm_ref, send_sems, recv_sems):
        my = lax.axis_index("i")
        left = lax.rem(my + N_DEV - 1, N_DEV)
        right = lax.rem(my + 1, N_DEV)

        barrier_sem = pltpu.get_barrier_semaphore()
        for nbr in (left, right):
            pl.semaphore_signal(
                barrier_sem, inc=1,
                device_id=(nbr,), device_id_type=pl.DeviceIdType.MESH,
            )
        pl.semaphore_wait(barrier_sem, 2)

        x2d = x_ref[...].reshape(B * SQ, D)
        q = jnp.dot(x2d, wq_ref[...], preferred_element_type=jnp.float32)
        k = jnp.dot(x2d, wk_ref[...], preferred_element_type=jnp.float32)
        v = jnp.dot(x2d, wv_ref[...], preferred_element_type=jnp.float32)

        for b in range(B):
            rows = pl.ds(b * SQ, SQ)
            for h in range(HQ_LOCAL):
                qbh = q[rows, h * DH:(h + 1) * DH]
                kv0 = (h // GROUP) * DH
                kbh = k[rows, kv0:kv0 + DH]
                vbh = v[rows, kv0:kv0 + DH]
                s = lax.dot_general(
                    qbh, kbh, (((1,), (1,)), ((), ())),
                    preferred_element_type=jnp.float32,
                ) * SCALE
                m = jnp.max(s, axis=1, keepdims=True)
                p = jnp.exp(s - m)
                l = jnp.sum(p, axis=1, keepdims=True)
                a = jnp.dot(p, vbh, preferred_element_type=jnp.float32) / l
                attn_ref[rows, h * DH:(h + 1) * DH] = a

        partial = jnp.dot(attn_ref[...], wo_ref[...],
                          preferred_element_type=jnp.float32)
        out_ref[...] = partial.reshape(B, SQ, D)
        comm_ref[0] = partial

        for hop in range(N_DEV - 1):
            rdma = pltpu.make_async_remote_copy(
                src_ref=comm_ref.at[hop],
                dst_ref=comm_ref.at[hop + 1],
                send_sem=send_sems.at[hop],
                recv_sem=recv_sems.at[hop],
                device_id=(right,),
                device_id_type=pl.DeviceIdType.MESH,
            )
            rdma.start()
            rdma.wait()
            out_ref[...] += comm_ref[hop + 1].reshape(B, SQ, D)

    return pl.pallas_call(
        body,
        out_shape=jax.ShapeDtypeStruct((B, SQ, D), jnp.float32),
        in_specs=[pl.BlockSpec(memory_space=pltpu.VMEM)] * 5,
        out_specs=pl.BlockSpec(memory_space=pltpu.VMEM),
        scratch_shapes=[
            pltpu.VMEM((B * SQ, D), jnp.float32),
            pltpu.VMEM((N_DEV, B * SQ, D), jnp.float32),
            pltpu.SemaphoreType.DMA((N_DEV - 1,)),
            pltpu.SemaphoreType.DMA((N_DEV - 1,)),
        ],
        compiler_params=pltpu.CompilerParams(collective_id=0),
    )(x, Wq, Wo, wk_loc, wv_loc)


# baseline (device time: 162550 ns/iter reference)
import jax
import jax.numpy as jnp
from jax import lax
from jax.experimental import pallas as pl
from jax.experimental.pallas import tpu as pltpu

N_DEV = 4
B, SQ, D = 4, 256, 1024
HQ_LOCAL, DH, GROUP = 8, 128, 4
KV_COLS = (HQ_LOCAL // GROUP) * DH
SCALE = 0.08838834764831843


def kernel(x, Wq, Wo, Wk, Wv):
    my_pos = lax.axis_index("i")
    wk_loc = lax.dynamic_slice(Wk, (0, my_pos * KV_COLS), (D, KV_COLS))
    wv_loc = lax.dynamic_slice(Wv, (0, my_pos * KV_COLS), (D, KV_COLS))

    def body(x_ref, wq_ref, wo_ref, wk_ref, wv_ref, out_ref,
             attn_ref, comm_ref, send_sems, recv_sems):
        my = lax.axis_index("i")
        left = lax.rem(my + N_DEV - 1, N_DEV)
        right = lax.rem(my + 1, N_DEV)

        barrier_sem = pltpu.get_barrier_semaphore()
        for nbr in (left, right):
            pl.semaphore_signal(
                barrier_sem, inc=1,
                device_id=(nbr,), device_id_type=pl.DeviceIdType.MESH,
            )
        pl.semaphore_wait(barrier_sem, 2)

        x2d = x_ref[...].reshape(B * SQ, D)
        q = jnp.dot(x2d, wq_ref[...], preferred_element_type=jnp.float32)
        k = jnp.dot(x2d, wk_ref[...], preferred_element_type=jnp.float32)
        v = jnp.dot(x2d, wv_ref[...], preferred_element_type=jnp.float32)

        for b in range(B):
            rows = slice(b * SQ, (b + 1) * SQ)
            for h in range(HQ_LOCAL):
                qbh = q[rows, h * DH:(h + 1) * DH]
                kv0 = (h // GROUP) * DH
                kbh = k[rows, kv0:kv0 + DH]
                vbh = v[rows, kv0:kv0 + DH]
                s = lax.dot_general(
                    qbh, kbh, (((1,), (1,)), ((), ())),
                    preferred_element_type=jnp.float32,
                ) * SCALE
                m = jnp.max(s, axis=1, keepdims=True)
                p = jnp.exp(s - m)
                l = jnp.sum(p, axis=1, keepdims=True)
                a = jnp.dot(p, vbh, preferred_element_type=jnp.float32) / l
                attn_ref[rows, h * DH:(h + 1) * DH] = a

        partial = jnp.dot(attn_ref[...], wo_ref[...],
                          preferred_element_type=jnp.float32)
        out_ref[...] = partial.reshape(B, SQ, D)
        comm_ref[0] = partial

        for hop in range(N_DEV - 1):
            rdma = pltpu.make_async_remote_copy(
                src_ref=comm_ref.at[hop],
                dst_ref=comm_ref.at[hop + 1],
                send_sem=send_sems.at[hop],
                recv_sem=recv_sems.at[hop],
                device_id=(right,),
                device_id_type=pl.DeviceIdType.MESH,
            )
            rdma.start()
            rdma.wait()
            out_ref[...] += comm_ref[hop + 1].reshape(B, SQ, D)

    return pl.pallas_call(
        body,
        out_shape=jax.ShapeDtypeStruct((B, SQ, D), jnp.float32),
        in_specs=[pl.BlockSpec(memory_space=pltpu.VMEM)] * 5,
        out_specs=pl.BlockSpec(memory_space=pltpu.VMEM),
        scratch_shapes=[
            pltpu.VMEM((B * SQ, D), jnp.float32),
            pltpu.VMEM((N_DEV, B * SQ, D), jnp.float32),
            pltpu.SemaphoreType.DMA((N_DEV - 1,)),
            pltpu.SemaphoreType.DMA((N_DEV - 1,)),
        ],
        compiler_params=pltpu.CompilerParams(collective_id=0),
    )(x, Wq, Wo, wk_loc, wv_loc)


# device time: 44744 ns/iter; 3.6329x vs baseline; 3.6329x over previous
import jax
import jax.numpy as jnp
from jax import lax
from jax.experimental import pallas as pl
from jax.experimental.pallas import tpu as pltpu

N_DEV = 4
B, SQ, D = 4, 256, 1024
HQ_LOCAL, DH, GROUP = 8, 128, 4
KV_COLS = (HQ_LOCAL // GROUP) * DH
SCALE = 0.08838834764831843
HALF = D // 2
R = B * SQ


def kernel(x, Wq, Wo, Wk, Wv):
    my_pos = lax.axis_index("i")
    wk_loc = lax.dynamic_slice(Wk, (0, my_pos * KV_COLS), (D, KV_COLS))
    wv_loc = lax.dynamic_slice(Wv, (0, my_pos * KV_COLS), (D, KV_COLS))

    def body(x_ref, wq_ref, wo_ref, wk_ref, wv_ref, out_ref,
             attn_ref, accA, accB, rA1, rB1, rA2, rB2,
             send_sems, recv_sems):
        my = lax.axis_index("i")
        p_m1 = my + 1 - 2 * lax.rem(my, 2)
        p_m2 = 3 - my
        hA = jnp.where(jnp.logical_or(my == 1, my == 2), 1, 0)
        qA = jnp.where(my >= 2, 1, 0)
        hB = jnp.where(my >= 2, 1, 0)
        qB = lax.rem(my, 2)

        barrier_sem = pltpu.get_barrier_semaphore()
        for nbr in (p_m1, p_m2):
            pl.semaphore_signal(
                barrier_sem, inc=1,
                device_id=(nbr,), device_id_type=pl.DeviceIdType.MESH,
            )
        pl.semaphore_wait(barrier_sem, 2)

        x2d = x_ref[...].reshape(R, D)
        q = jnp.dot(x2d, wq_ref[...], preferred_element_type=jnp.float32)
        k = jnp.dot(x2d, wk_ref[...], preferred_element_type=jnp.float32)
        v = jnp.dot(x2d, wv_ref[...], preferred_element_type=jnp.float32)

        for b in range(B):
            rows = slice(b * SQ, (b + 1) * SQ)
            for h in range(HQ_LOCAL):
                qbh = q[rows, h * DH:(h + 1) * DH]
                kv0 = (h // GROUP) * DH
                kbh = k[rows, kv0:kv0 + DH]
                vbh = v[rows, kv0:kv0 + DH]
                s = lax.dot_general(
                    qbh, kbh, (((1,), (1,)), ((), ())),
                    preferred_element_type=jnp.float32,
                ) * SCALE
                m = jnp.max(s, axis=1, keepdims=True)
                p = jnp.exp(s - m)
                l = jnp.sum(p, axis=1, keepdims=True)
                a = jnp.dot(p, vbh, preferred_element_type=jnp.float32) / l
                attn_ref[rows, h * DH:(h + 1) * DH] = a

        partial = jnp.dot(attn_ref[...], wo_ref[...],
                          preferred_element_type=jnp.float32)
        accA[...] = partial[:, :HALF].astype(jnp.bfloat16)
        accB[...] = partial[:, HALF:].astype(jnp.bfloat16)

        def exchange(src, dst, partner, idx):
            rdma = pltpu.make_async_remote_copy(
                src_ref=src, dst_ref=dst,
                send_sem=send_sems.at[idx], recv_sem=recv_sems.at[idx],
                device_id=(partner,), device_id_type=pl.DeviceIdType.MESH,
            )
            rdma.start()
            return rdma

        ea = exchange(accA.at[pl.ds((1 - hA) * 512, 512)], rA1, p_m1, 0)
        eb = exchange(accB.at[pl.ds((1 - hB) * 512, 512)], rB1, p_m2, 1)
        ea.wait()
        accA[pl.ds(hA * 512, 512), :] += rA1[...]
        eb.wait()
        accB[pl.ds(hB * 512, 512), :] += rB1[...]

        offA = hA * 512 + qA * 256
        offB = hB * 512 + qB * 256
        ea = exchange(accA.at[pl.ds(hA * 512 + (1 - qA) * 256, 256)],
                      rA2, p_m2, 2)
        eb = exchange(accB.at[pl.ds(hB * 512 + (1 - qB) * 256, 256)],
                      rB2, p_m1, 3)
        ea.wait()
        accA[pl.ds(offA, 256), :] += rA2[...]
        eb.wait()
        accB[pl.ds(offB, 256), :] += rB2[...]

        ea = exchange(accA.at[pl.ds(offA, 256)],
                      accA.at[pl.ds(offA, 256)], p_m2, 4)
        eb = exchange(accB.at[pl.ds(offB, 256)],
                      accB.at[pl.ds(offB, 256)], p_m1, 5)
        ea.wait()
        eb.wait()

        ea = exchange(accA.at[pl.ds(hA * 512, 512)],
                      accA.at[pl.ds(hA * 512, 512)], p_m1, 6)
        eb = exchange(accB.at[pl.ds(hB * 512, 512)],
                      accB.at[pl.ds(hB * 512, 512)], p_m2, 7)
        ea.wait()
        eb.wait()

        out_ref[:, :, :HALF] = accA[...].astype(jnp.float32).reshape(B, SQ, HALF)
        out_ref[:, :, HALF:] = accB[...].astype(jnp.float32).reshape(B, SQ, HALF)

    return pl.pallas_call(
        body,
        out_shape=jax.ShapeDtypeStruct((B, SQ, D), jnp.float32),
        in_specs=[pl.BlockSpec(memory_space=pltpu.VMEM)] * 5,
        out_specs=pl.BlockSpec(memory_space=pltpu.VMEM),
        scratch_shapes=[
            pltpu.VMEM((R, D), jnp.float32),
            pltpu.VMEM((R, HALF), jnp.bfloat16),
            pltpu.VMEM((R, HALF), jnp.bfloat16),
            pltpu.VMEM((512, HALF), jnp.bfloat16),
            pltpu.VMEM((512, HALF), jnp.bfloat16),
            pltpu.VMEM((256, HALF), jnp.bfloat16),
            pltpu.VMEM((256, HALF), jnp.bfloat16),
            pltpu.SemaphoreType.DMA((8,)),
            pltpu.SemaphoreType.DMA((8,)),
        ],
        compiler_params=pltpu.CompilerParams(collective_id=0),
    )(x, Wq, Wo, wk_loc, wv_loc)


# device time: 39310 ns/iter; 4.1351x vs baseline; 1.1382x over previous
import jax
import jax.numpy as jnp
from jax import lax
from jax.experimental import pallas as pl
from jax.experimental.pallas import tpu as pltpu

N_DEV = 4
B, SQ, D = 4, 256, 1024
HQ_LOCAL, DH, GROUP = 8, 128, 4
KV_COLS = (HQ_LOCAL // GROUP) * DH
SCALE = 0.08838834764831843
HALF = D // 2
R = B * SQ


def kernel(x, Wq, Wo, Wk, Wv):
    my_pos = lax.axis_index("i")
    wk_loc = lax.dynamic_slice(Wk, (0, my_pos * KV_COLS), (D, KV_COLS))
    wv_loc = lax.dynamic_slice(Wv, (0, my_pos * KV_COLS), (D, KV_COLS))

    def body(x_ref, wq_ref, wo_ref, wk_ref, wv_ref, out_ref,
             accA, accB,
             r1_pa, r1_pb, r1_qa, r1_qb,
             r2_pa, r2_pb, r2_qa, r2_qb,
             send_sems, recv_sems):
        my = lax.axis_index("i")
        p_m1 = my + 1 - 2 * lax.rem(my, 2)
        p_m2 = 3 - my
        h12 = jnp.where(jnp.logical_or(my == 1, my == 2), 1, 0)
        q12 = jnp.where(my >= 2, 1, 0)
        h21 = jnp.where(my >= 2, 1, 0)
        q21 = lax.rem(my, 2)

        barrier_sem = pltpu.get_barrier_semaphore()
        for nbr in (p_m1, p_m2):
            pl.semaphore_signal(
                barrier_sem, inc=1,
                device_id=(nbr,), device_id_type=pl.DeviceIdType.MESH,
            )
        pl.semaphore_wait(barrier_sem, 2)

        def compute_batch(b):
            xb = x_ref[b]
            qb = jnp.dot(xb, wq_ref[...], preferred_element_type=jnp.float32)
            kb = jnp.dot(xb, wk_ref[...], preferred_element_type=jnp.float32)
            vb = jnp.dot(xb, wv_ref[...], preferred_element_type=jnp.float32)
            heads = []
            for h in range(HQ_LOCAL):
                qh = qb[:, h * DH:(h + 1) * DH]
                kv0 = (h // GROUP) * DH
                kh = kb[:, kv0:kv0 + DH]
                vh = vb[:, kv0:kv0 + DH]
                s = lax.dot_general(
                    qh, kh, (((1,), (1,)), ((), ())),
                    preferred_element_type=jnp.float32,
                ) * SCALE
                m = jnp.max(s, axis=1, keepdims=True)
                p = jnp.exp(s - m)
                l = jnp.sum(p, axis=1, keepdims=True)
                heads.append(
                    jnp.dot(p, vh, preferred_element_type=jnp.float32) / l)
            attn = jnp.concatenate(heads, axis=1)
            partial = jnp.dot(attn, wo_ref[...],
                              preferred_element_type=jnp.float32)
            rows = slice(b * SQ, (b + 1) * SQ)
            accA[rows, :] = partial[:, :HALF].astype(jnp.bfloat16)
            accB[rows, :] = partial[:, HALF:].astype(jnp.bfloat16)

        def exchange(src, dst, partner, idx):
            rdma = pltpu.make_async_remote_copy(
                src_ref=src, dst_ref=dst,
                send_sem=send_sems.at[idx], recv_sem=recv_sems.at[idx],
                device_id=(partner,), device_id_type=pl.DeviceIdType.MESH,
            )
            rdma.start()
            return rdma

        def st(acc, rb, pf, ps, h, q, r1, r2, s0):
            return dict(acc=acc, rb=rb, pf=pf, ps=ps, h=h, q=q,
                        r1=r1, r2=r2, s0=s0)

        PA = st(accA, 0,   p_m1, p_m2, h12, q12, r1_pa, r2_pa, 0)
        PB = st(accB, 0,   p_m2, p_m1, h21, q21, r1_pb, r2_pb, 4)
        QA = st(accA, 512, p_m2, p_m1, h21, q21, r1_qa, r2_qa, 8)
        QB = st(accB, 512, p_m1, p_m2, h12, q12, r1_qb, r2_qb, 12)

        def t1_start(S):
            src = S["acc"].at[pl.ds(S["rb"] + (1 - S["h"]) * 256, 256)]
            return exchange(src, S["r1"], S["pf"], S["s0"] + 0)

        def t1_finish(S):
            S["acc"][pl.ds(S["rb"] + S["h"] * 256, 256), :] += S["r1"][...]

        def t2_start(S):
            off = S["rb"] + S["h"] * 256 + (1 - S["q"]) * 128
            return exchange(S["acc"].at[pl.ds(off, 128)], S["r2"],
                            S["ps"], S["s0"] + 1)

        def t2_finish(S):
            off = S["rb"] + S["h"] * 256 + S["q"] * 128
            S["acc"][pl.ds(off, 128), :] += S["r2"][...]

        def t3_start(S):
            off = S["rb"] + S["h"] * 256 + S["q"] * 128
            sl = S["acc"].at[pl.ds(off, 128)]
            return exchange(sl, sl, S["ps"], S["s0"] + 2)

        def t4_start(S):
            sl = S["acc"].at[pl.ds(S["rb"] + S["h"] * 256, 256)]
            return exchange(sl, sl, S["pf"], S["s0"] + 3)

        compute_batch(0)
        compute_batch(1)
        p_t1 = [t1_start(PA), t1_start(PB)]
        compute_batch(2)
        compute_batch(3)
        q_t1 = [t1_start(QA), t1_start(QB)]

        for e in p_t1:
            e.wait_recv()
        t1_finish(PA); t1_finish(PB)
        p_t2 = [t2_start(PA), t2_start(PB)]

        for e in q_t1:
            e.wait_recv()
        t1_finish(QA); t1_finish(QB)
        q_t2 = [t2_start(QA), t2_start(QB)]

        for e in p_t2:
            e.wait_recv()
        t2_finish(PA); t2_finish(PB)
        p_t3 = [t3_start(PA), t3_start(PB)]

        for e in q_t2:
            e.wait_recv()
        t2_finish(QA); t2_finish(QB)
        q_t3 = [t3_start(QA), t3_start(QB)]

        for e in p_t3:
            e.wait_recv()
        p_t4 = [t4_start(PA), t4_start(PB)]

        for e in q_t3:
            e.wait_recv()
        q_t4 = [t4_start(QA), t4_start(QB)]

        for e in p_t4:
            e.wait_recv()
        out_ref[0:2, :, :HALF] = (
            accA[0:512, :].astype(jnp.float32).reshape(2, SQ, HALF))
        out_ref[0:2, :, HALF:] = (
            accB[0:512, :].astype(jnp.float32).reshape(2, SQ, HALF))

        for e in q_t4:
            e.wait_recv()
        out_ref[2:4, :, :HALF] = (
            accA[512:1024, :].astype(jnp.float32).reshape(2, SQ, HALF))
        out_ref[2:4, :, HALF:] = (
            accB[512:1024, :].astype(jnp.float32).reshape(2, SQ, HALF))

        for e in p_t1 + q_t1 + p_t2 + q_t2 + p_t3 + q_t3 + p_t4 + q_t4:
            e.wait_send()

    return pl.pallas_call(
        body,
        out_shape=jax.ShapeDtypeStruct((B, SQ, D), jnp.float32),
        in_specs=[pl.BlockSpec(memory_space=pltpu.VMEM)] * 5,
        out_specs=pl.BlockSpec(memory_space=pltpu.VMEM),
        scratch_shapes=[
            pltpu.VMEM((R, HALF), jnp.bfloat16),
            pltpu.VMEM((R, HALF), jnp.bfloat16),
            pltpu.VMEM((256, HALF), jnp.bfloat16),
            pltpu.VMEM((256, HALF), jnp.bfloat16),
            pltpu.VMEM((256, HALF), jnp.bfloat16),
            pltpu.VMEM((256, HALF), jnp.bfloat16),
            pltpu.VMEM((128, HALF), jnp.bfloat16),
            pltpu.VMEM((128, HALF), jnp.bfloat16),
            pltpu.VMEM((128, HALF), jnp.bfloat16),
            pltpu.VMEM((128, HALF), jnp.bfloat16),
            pltpu.SemaphoreType.DMA((16,)),
            pltpu.SemaphoreType.DMA((16,)),
        ],
        compiler_params=pltpu.CompilerParams(collective_id=0),
    )(x, Wq, Wo, wk_loc, wv_loc)
